# baseline (speedup 1.0000x reference)
import functools

import jax
import jax.numpy as jnp
from jax import lax
from jax.experimental import pallas as pl
from jax.experimental.pallas import tpu as pltpu
from jax.experimental.pallas import tpu_sc as plsc

_V = 1000
_D = 128
_B = 4096
_T = 200
_NW = 32
_ROWS_PER_W = (_B * _T) // _NW
_CHUNK = 128
_NCHUNK = _ROWS_PER_W // _CHUNK
_NBUF = 5
_LAG = 2
_HBM_SLOT = -1


def _emb_body(idx_hbm, table_hbm, out_hbm, tbl_sh, idx_v, rows_v, gsem, hsem, wsem):
    cid = lax.axis_index("c")
    sid = lax.axis_index("s")
    wid = sid * 2 + cid
    out_base = wid * _ROWS_PER_W

    @pl.when(sid == 0)
    def _():
        pltpu.sync_copy(table_hbm, tbl_sh)

    plsc.subcore_barrier()

    pltpu.sync_copy(idx_hbm.at[wid], idx_v)

    def gather_issue(i, j):
        if j == _HBM_SLOT:
            pltpu.async_copy(table_hbm.at[idx_v.at[i]], rows_v.at[j], hsem)
        else:
            pltpu.async_copy(tbl_sh.at[idx_v.at[i]], rows_v.at[j], gsem)

    def gather_drain(j):
        if j == _HBM_SLOT:
            pltpu.make_async_copy(
                table_hbm.at[pl.ds(0, _CHUNK)], rows_v.at[j], hsem
            ).wait()
        else:
            pltpu.make_async_copy(
                tbl_sh.at[pl.ds(0, _CHUNK)], rows_v.at[j], gsem
            ).wait()

    def wb_issue(g, j):
        pltpu.async_copy(
            rows_v.at[j], out_hbm.at[pl.ds(out_base + g * _CHUNK, _CHUNK)], wsem
        )

    def wb_drain(j):
        pltpu.make_async_copy(
            rows_v.at[j], out_hbm.at[pl.ds(out_base, _CHUNK)], wsem
        ).wait()

    for i in range(_NBUF):
        gather_issue(i, i)
        if i >= _LAG:
            g = i - _LAG
            gather_drain(g % _NBUF)
            wb_issue(g, g % _NBUF)

    def outer(o, carry):
        for j in range(_NBUF):
            i = _NBUF + o * _NBUF + j
            wb_drain(j)
            gather_issue(i, j)
            jg = (j - _LAG) % _NBUF
            gather_drain(jg)
            wb_issue(i - _LAG, jg)
        return carry

    lax.fori_loop(0, (_NCHUNK - _NBUF) // _NBUF, outer, 0)

    for g in range(_NCHUNK - _LAG, _NCHUNK):
        gather_drain(g % _NBUF)
        wb_issue(g, g % _NBUF)
    for j in range(_NBUF):
        wb_drain(j)


_emb = functools.partial(
    pl.kernel,
    mesh=plsc.VectorSubcoreMesh(core_axis_name="c", subcore_axis_name="s"),
    out_type=jax.ShapeDtypeStruct((_B * _T, _D), jnp.float32),
    scratch_types=[
        pltpu.MemorySpace.VMEM_SHARED((_V, _D), jnp.float32),
        pltpu.VMEM((_NCHUNK, _CHUNK), jnp.int32),
        pltpu.VMEM((_NBUF, _CHUNK, _D), jnp.float32),
        pltpu.SemaphoreType.DMA,
        pltpu.SemaphoreType.DMA,
        pltpu.SemaphoreType.DMA,
    ],
)(_emb_body)


def kernel(vocab_ids, table):
    idx = vocab_ids.reshape(_NW, _NCHUNK, _CHUNK).astype(jnp.int32)
    out = _emb(idx, table)
    return out.reshape(_B, _T, _D)


# speedup vs baseline: 1.0012x; 1.0012x over previous
import functools

import jax
import jax.numpy as jnp
from jax import lax
from jax.experimental import pallas as pl
from jax.experimental.pallas import tpu as pltpu
from jax.experimental.pallas import tpu_sc as plsc

_V = 1000
_D = 128
_B = 4096
_T = 200
_NW = 32
_ROWS_PER_W = (_B * _T) // _NW
_CHUNK = 128
_NCHUNK = _ROWS_PER_W // _CHUNK
_NBUF = 4
_LAG = 2


def _emb_body(idx_hbm, table_hbm, out_hbm, tbl_sh, idx_v, rows_v, gsem, wsem):
    cid = lax.axis_index("c")
    sid = lax.axis_index("s")
    wid = sid * 2 + cid
    out_base = wid * _ROWS_PER_W

    @pl.when(sid == 0)
    def _():
        pltpu.sync_copy(table_hbm, tbl_sh)

    plsc.subcore_barrier()

    pltpu.sync_copy(idx_hbm.at[wid], idx_v)

    def gather_issue(i, j):
        pltpu.async_copy(tbl_sh.at[idx_v.at[i]], rows_v.at[j], gsem)

    def gather_drain(j):
        pltpu.make_async_copy(tbl_sh.at[pl.ds(0, _CHUNK)], rows_v.at[j], gsem).wait()

    def wb_issue(g, j):
        pltpu.async_copy(
            rows_v.at[j], out_hbm.at[pl.ds(out_base + g * _CHUNK, _CHUNK)], wsem
        )

    def wb_drain(j):
        pltpu.make_async_copy(
            rows_v.at[j], out_hbm.at[pl.ds(out_base, _CHUNK)], wsem
        ).wait()

    for i in range(_NBUF):
        gather_issue(i, i)
        if i >= _LAG:
            g = i - _LAG
            gather_drain(g % _NBUF)
            wb_issue(g, g % _NBUF)

    def outer(o, carry):
        for j in range(_NBUF):
            i = _NBUF + o * _NBUF + j
            wb_drain(j)
            gather_issue(i, j)
            jg = (j - _LAG) % _NBUF
            gather_drain(jg)
            wb_issue(i - _LAG, jg)
        return carry

    lax.fori_loop(0, (_NCHUNK - _NBUF) // _NBUF, outer, 0)

    for g in range(_NCHUNK - _LAG, _NCHUNK):
        gather_drain(g % _NBUF)
        wb_issue(g, g % _NBUF)
    for j in range(_NBUF):
        wb_drain(j)


_emb = functools.partial(
    pl.kernel,
    mesh=plsc.VectorSubcoreMesh(core_axis_name="c", subcore_axis_name="s"),
    out_type=jax.ShapeDtypeStruct((_B * _T, _D), jnp.float32),
    scratch_types=[
        pltpu.MemorySpace.VMEM_SHARED((_V, _D), jnp.float32),
        pltpu.VMEM((_NCHUNK, _CHUNK), jnp.int32),
        pltpu.VMEM((_NBUF, _CHUNK, _D), jnp.float32),
        pltpu.SemaphoreType.DMA,
        pltpu.SemaphoreType.DMA,
    ],
)(_emb_body)


def kernel(vocab_ids, table):
    idx = vocab_ids.reshape(_NW, _NCHUNK, _CHUNK).astype(jnp.int32)
    out = _emb(idx, table)
    return out.reshape(_B, _T, _D)
